# baseline (speedup 1.0000x reference)
import functools

import jax
import jax.numpy as jnp
from jax import lax
from jax.experimental import pallas as pl
from jax.experimental.pallas import tpu as pltpu
from jax.experimental.pallas import tpu_sc as plsc

N = 10000
E = 320000
D = 128
DE = 16
DO = 128

NC = 2
NS = 16
NW = NC * NS
CH = 128
FULL = (E // CH) // NW
EPW = FULL * CH
TAILW = E // CH - FULL * NW
TAILB = FULL * NW * CH
MAXCH = FULL + 1
NP = 10240
NSTR = NP // NS
NPA = 10112
NSTRA = NPA // NS
NBUF = 3

_mesh = plsc.VectorSubcoreMesh(core_axis_name="c", subcore_axis_name="s")


def _nchunks(w):
    return FULL + jnp.where(w < TAILW, 1, 0)


def _ebase(w, j):
    return jnp.where(j < FULL, w * EPW + j * CH, TAILB + w * CH)


def _fill_1d(ref, n, value):
    vec = jnp.full((16,), value, jnp.float32)

    def body(i, _):
        ref[pl.ds(i * 16, 16)] = vec
        return 0

    lax.fori_loop(0, n // 16, body, 0)


def _copy_row(big, off, row):
    for k in range(CH // 16):
        row[pl.ds(k * 16, 16)] = big[pl.ds(off + k * 16, 16)]


@functools.partial(
    pl.kernel,
    out_type=[
        jax.ShapeDtypeStruct((E,), jnp.int32),
        jax.ShapeDtypeStruct((E,), jnp.int32),
    ],
    mesh=_mesh,
    scratch_types=[
        pltpu.VMEM((2, EPW), jnp.int32),
        pltpu.VMEM((2, CH), jnp.int32),
    ],
    compiler_params=pltpu.CompilerParams(use_tc_tiling_on_sc=True),
)
def _sc_split(ei_hbm, src_out, dst_out, buf_v, tbuf_v):
    c = lax.axis_index("c")
    s = lax.axis_index("s")
    w = c * NS + s

    pltpu.sync_copy(ei_hbm.at[:, pl.ds(w * EPW, EPW)], buf_v)
    pltpu.sync_copy(buf_v.at[0], src_out.at[pl.ds(w * EPW, EPW)])
    pltpu.sync_copy(buf_v.at[1], dst_out.at[pl.ds(w * EPW, EPW)])

    @pl.when(w < TAILW)
    def _():
        pltpu.sync_copy(ei_hbm.at[:, pl.ds(TAILB + w * CH, CH)], tbuf_v)
        pltpu.sync_copy(tbuf_v.at[0], src_out.at[pl.ds(TAILB + w * CH, CH)])
        pltpu.sync_copy(tbuf_v.at[1], dst_out.at[pl.ds(TAILB + w * CH, CH)])


@functools.partial(
    pl.kernel,
    out_type=[
        jax.ShapeDtypeStruct((NC, 1, NP), jnp.float32),
        jax.ShapeDtypeStruct((NC, 1, NP), jnp.float32),
        jax.ShapeDtypeStruct((NC, NP, DE), jnp.float32),
    ],
    mesh=_mesh,
    scratch_types=[
        pltpu.VMEM((MAXCH * CH,), jnp.int32),
        pltpu.VMEM((MAXCH * CH,), jnp.int32),
        pltpu.VMEM((CH,), jnp.int32),
        pltpu.VMEM((CH,), jnp.int32),
        pltpu.VMEM((2, CH * DE // 128, 128), jnp.float32),
        pltpu.VMEM((CH, DE), jnp.float32),
        pltpu.VMEM((NSTR, DE), jnp.float32),
        pltpu.VMEM((NSTR,), jnp.float32),
        pltpu.VMEM((CH,), jnp.float32),
        pltpu.VMEM_SHARED((NP,), jnp.float32),
        pltpu.VMEM_SHARED((NP,), jnp.float32),
        pltpu.VMEM_SHARED((NP, DE), jnp.float32),
        pltpu.SemaphoreType.DMA,
        pltpu.SemaphoreType.DMA,
    ],
    compiler_params=pltpu.CompilerParams(use_tc_tiling_on_sc=False),
)
def _sc_edge(src1d, dst1d, ef128, outdeg_out, indeg_out, agge_out,
             src_v, dst_v, srow_v, drow_v, ef_v, eft_v, ze_v, zst_v, ones_v,
             hs_sh, hd_sh, agge_sh, esem0, esem1):
    c = lax.axis_index("c")
    s = lax.axis_index("s")
    w = c * NS + s
    esems = (esem0, esem1)

    def zb(i, _):
        ze_v[i, :] = jnp.zeros((DE,), jnp.float32)
        return 0

    lax.fori_loop(0, NSTR, zb, 0)
    _fill_1d(zst_v, NSTR, 0.0)
    _fill_1d(ones_v, CH, 1.0)
    base = s * NSTR
    pltpu.sync_copy(zst_v, hs_sh.at[pl.ds(base, NSTR)])
    pltpu.sync_copy(zst_v, hd_sh.at[pl.ds(base, NSTR)])
    pltpu.sync_copy(ze_v, agge_sh.at[pl.ds(base, NSTR)])
    plsc.subcore_barrier()

    pltpu.sync_copy(src1d.at[pl.ds(w * EPW, EPW)], src_v.at[pl.ds(0, EPW)])
    pltpu.sync_copy(dst1d.at[pl.ds(w * EPW, EPW)], dst_v.at[pl.ds(0, EPW)])
    nchunks = _nchunks(w)

    @pl.when(w < TAILW)
    def _():
        pltpu.sync_copy(src1d.at[pl.ds(TAILB + w * CH, CH)],
                        src_v.at[pl.ds(EPW, CH)])
        pltpu.sync_copy(dst1d.at[pl.ds(TAILB + w * CH, CH)],
                        dst_v.at[pl.ds(EPW, CH)])

    EFR = CH * DE // 128

    def issue(j, b):
        pltpu.async_copy(ef128.at[pl.ds(_ebase(w, j) // 8, EFR)],
                         ef_v.at[b], esems[b])

    def repack_ef(b):
        def rp(r, _):
            for kk in range(8):
                eft_v[8 * r + kk, :] = ef_v[b, r, pl.ds(16 * kk, 16)]
            return 0

        lax.fori_loop(0, EFR, rp, 0)

    issue(0, 0)

    def g_body(g, _):
        for b in range(2):
            j = 2 * g + b
            nb = 1 - b

            @pl.when(j < nchunks)
            def _():
                @pl.when(j + 1 < nchunks)
                def _():
                    issue(j + 1, nb)

                _copy_row(src_v, j * CH, srow_v)
                _copy_row(dst_v, j * CH, drow_v)
                pltpu.sync_copy(ones_v, hs_sh.at[srow_v], add=True)
                pltpu.sync_copy(ones_v, hd_sh.at[drow_v], add=True)
                pltpu.make_async_copy(
                    ef128.at[pl.ds(0, EFR)], ef_v.at[b], esems[b]).wait()
                repack_ef(b)
                pltpu.sync_copy(eft_v, agge_sh.at[drow_v], add=True)
        return 0

    lax.fori_loop(0, MAXCH // 2 + 1, g_body, 0)
    plsc.subcore_barrier()

    pltpu.sync_copy(hs_sh.at[pl.ds(base, NSTR)],
                    outdeg_out.at[c, 0, pl.ds(base, NSTR)])
    pltpu.sync_copy(hd_sh.at[pl.ds(base, NSTR)],
                    indeg_out.at[c, 0, pl.ds(base, NSTR)])
    for k in range(NSTR // CH):
        pltpu.sync_copy(agge_sh.at[pl.ds(base + k * CH, CH)],
                        agge_out.at[c, pl.ds(base + k * CH, CH)])


@functools.partial(
    pl.kernel,
    out_type=jax.ShapeDtypeStruct((NC, NPA, D), jnp.float32),
    mesh=_mesh,
    scratch_types=[
        pltpu.VMEM((CH,), jnp.int32),
        pltpu.VMEM((CH,), jnp.int32),
        pltpu.VMEM((CH,), jnp.int32),
        pltpu.VMEM((CH,), jnp.int32),
        pltpu.VMEM((CH,), jnp.int32),
        pltpu.VMEM((CH,), jnp.int32),
        pltpu.VMEM((NBUF, CH, D), jnp.float32),
        pltpu.VMEM_SHARED((NPA, D), jnp.float32),
        pltpu.SemaphoreType.DMA,
        pltpu.SemaphoreType.DMA,
        pltpu.SemaphoreType.DMA,
        pltpu.SemaphoreType.DMA,
        pltpu.SemaphoreType.DMA,
        pltpu.SemaphoreType.DMA,
        pltpu.SemaphoreType.DMA,
        pltpu.SemaphoreType.DMA,
        pltpu.SemaphoreType.DMA,
    ],
)
def _sc_aggh(h_hbm, src1d, dst1d, aggh_out,
             sr0, sr1, sr2, dr0, dr1, dr2, rows_v, aggh_sh,
             is0, is1, is2, id0, id1, id2, g0, g1, g2):
    c = lax.axis_index("c")
    s = lax.axis_index("s")
    w = c * NS + s
    srs = (sr0, sr1, sr2)
    drs = (dr0, dr1, dr2)
    isems = (is0, is1, is2)
    idsems = (id0, id1, id2)
    gsems = (g0, g1, g2)

    def zb(i, _):
        for k in range(D // 16):
            rows_v[0, i, pl.ds(k * 16, 16)] = jnp.zeros((16,), jnp.float32)
        return 0

    lax.fori_loop(0, CH, zb, 0)
    base = s * NSTRA
    for k in range(NSTRA // CH):
        pltpu.sync_copy(rows_v.at[0], aggh_sh.at[pl.ds(base + k * CH, CH)])
    rem = NSTRA - (NSTRA // CH) * CH
    pltpu.sync_copy(rows_v.at[0, pl.ds(0, rem)],
                    aggh_sh.at[pl.ds(base + NSTRA - rem, rem)])
    plsc.subcore_barrier()

    nchunks = _nchunks(w)

    def issue_idx(j, b):
        eb = _ebase(w, j)
        pltpu.async_copy(src1d.at[pl.ds(eb, CH)], srs[b], isems[b])
        pltpu.async_copy(dst1d.at[pl.ds(eb, CH)], drs[b], idsems[b])

    def wait_idx(b):
        pltpu.make_async_copy(
            src1d.at[pl.ds(0, CH)], srs[b], isems[b]).wait()
        pltpu.make_async_copy(
            dst1d.at[pl.ds(0, CH)], drs[b], idsems[b]).wait()

    def issue_g(b):
        pltpu.async_copy(h_hbm.at[srs[b]], rows_v.at[b], gsems[b])

    def wait_g(b):
        pltpu.make_async_copy(
            h_hbm.at[srs[b]], rows_v.at[b], gsems[b]).wait()

    issue_idx(0, 0)
    wait_idx(0)
    issue_g(0)

    @pl.when(1 < nchunks)
    def _():
        issue_idx(1, 1)
        wait_idx(1)
        issue_g(1)

    @pl.when(2 < nchunks)
    def _():
        issue_idx(2, 2)

    def g_body(g, _):
        for b in range(NBUF):
            j = NBUF * g + b
            nxt = (b + 2) % NBUF

            @pl.when(j < nchunks)
            def _():
                @pl.when(j + 2 < nchunks)
                def _():
                    wait_idx(nxt)
                    issue_g(nxt)

                wait_g(b)
                pltpu.sync_copy(rows_v.at[b], aggh_sh.at[drs[b]], add=True)

                @pl.when(j + NBUF < nchunks)
                def _():
                    issue_idx(j + NBUF, b)
        return 0

    lax.fori_loop(0, (MAXCH + NBUF - 1) // NBUF, g_body, 0)
    plsc.subcore_barrier()

    for k in range(NSTRA // CH):
        pltpu.sync_copy(aggh_sh.at[pl.ds(base + k * CH, CH)],
                        aggh_out.at[c, pl.ds(base + k * CH, CH)])
    pltpu.sync_copy(aggh_sh.at[pl.ds(base + NSTRA - rem, rem)],
                    aggh_out.at[c, pl.ds(base + NSTRA - rem, rem)])


def _prep_body(outdeg_ref, feat_ref, h_ref):
    deg = outdeg_ref[0, 0, :N] + outdeg_ref[1, 0, :N]
    norm = lax.rsqrt(jnp.maximum(deg, 1.0))
    h_ref[...] = feat_ref[...] * norm[:, None]


def _final_body(aggh_ref, agge_ref, wh_ref, we_ref, bias_ref, indeg_ref,
                out_ref):
    aggh = aggh_ref[0, :N] + aggh_ref[1, :N]
    agge = agge_ref[0, :N] + agge_ref[1, :N]
    acc = jnp.dot(aggh, wh_ref[...], preferred_element_type=jnp.float32)
    acc = acc + jnp.dot(agge, we_ref[...], preferred_element_type=jnp.float32)
    indeg = indeg_ref[0, 0, :N] + indeg_ref[1, 0, :N]
    norm = lax.rsqrt(jnp.maximum(indeg, 1.0))
    out_ref[...] = acc * norm[:, None] + bias_ref[...]


@jax.jit
def kernel(feat, edge_index, edge_feat, weight, bias):
    src1d, dst1d = _sc_split(edge_index)
    ef128 = edge_feat.reshape(E * DE // 128, 128)
    outdeg_p, indeg_p, agge_p = _sc_edge(src1d, dst1d, ef128)

    h = pl.pallas_call(
        _prep_body,
        out_shape=jax.ShapeDtypeStruct((N, D), jnp.float32),
    )(outdeg_p, feat)

    aggh_p = _sc_aggh(h, src1d, dst1d)

    out = pl.pallas_call(
        _final_body,
        out_shape=jax.ShapeDtypeStruct((N, DO), jnp.float32),
    )(aggh_p, agge_p, weight[:D], weight[D:], bias.reshape(1, DO), indeg_p)
    return out


# speedup vs baseline: 1.1259x; 1.1259x over previous
import functools

import jax
import jax.numpy as jnp
from jax import lax
from jax.experimental import pallas as pl
from jax.experimental.pallas import tpu as pltpu
from jax.experimental.pallas import tpu_sc as plsc

N = 10000
E = 320000
D = 128
DE = 16
DO = 128

NC = 2
NS = 16
NW = NC * NS
CH = 128
FULL = (E // CH) // NW
EPW = FULL * CH
TAILW = E // CH - FULL * NW
TAILB = FULL * NW * CH
MAXCH = FULL + 1
NP = 10240
NSTR = NP // NS
NPA = 10112
NSTRA = NPA // NS
NBUF = 3

_mesh = plsc.VectorSubcoreMesh(core_axis_name="c", subcore_axis_name="s")


def _nchunks(w):
    return FULL + jnp.where(w < TAILW, 1, 0)


def _ebase(w, j):
    return jnp.where(j < FULL, w * EPW + j * CH, TAILB + w * CH)


def _fill_1d(ref, n, value):
    vec = jnp.full((16,), value, jnp.float32)

    def body(i, _):
        ref[pl.ds(i * 16, 16)] = vec
        return 0

    lax.fori_loop(0, n // 16, body, 0)


def _copy_row(big, off, row):
    for k in range(CH // 16):
        row[pl.ds(k * 16, 16)] = big[pl.ds(off + k * 16, 16)]


@functools.partial(
    pl.kernel,
    out_type=[
        jax.ShapeDtypeStruct((E,), jnp.int32),
        jax.ShapeDtypeStruct((E,), jnp.int32),
    ],
    mesh=_mesh,
    scratch_types=[
        pltpu.VMEM((2, EPW), jnp.int32),
        pltpu.VMEM((2, CH), jnp.int32),
    ],
    compiler_params=pltpu.CompilerParams(use_tc_tiling_on_sc=True),
)
def _sc_split(ei_hbm, src_out, dst_out, buf_v, tbuf_v):
    c = lax.axis_index("c")
    s = lax.axis_index("s")
    w = c * NS + s

    pltpu.sync_copy(ei_hbm.at[:, pl.ds(w * EPW, EPW)], buf_v)
    pltpu.sync_copy(buf_v.at[0], src_out.at[pl.ds(w * EPW, EPW)])
    pltpu.sync_copy(buf_v.at[1], dst_out.at[pl.ds(w * EPW, EPW)])

    @pl.when(w < TAILW)
    def _():
        pltpu.sync_copy(ei_hbm.at[:, pl.ds(TAILB + w * CH, CH)], tbuf_v)
        pltpu.sync_copy(tbuf_v.at[0], src_out.at[pl.ds(TAILB + w * CH, CH)])
        pltpu.sync_copy(tbuf_v.at[1], dst_out.at[pl.ds(TAILB + w * CH, CH)])


@functools.partial(
    pl.kernel,
    out_type=[
        jax.ShapeDtypeStruct((NC, 1, NP), jnp.float32),
        jax.ShapeDtypeStruct((NC, 1, NP), jnp.float32),
    ],
    mesh=_mesh,
    scratch_types=[
        pltpu.VMEM((MAXCH * CH,), jnp.int32),
        pltpu.VMEM((MAXCH * CH,), jnp.int32),
        pltpu.VMEM((CH,), jnp.int32),
        pltpu.VMEM((CH,), jnp.int32),
        pltpu.VMEM((NSTR,), jnp.float32),
        pltpu.VMEM((CH,), jnp.float32),
        pltpu.VMEM_SHARED((NP,), jnp.float32),
        pltpu.VMEM_SHARED((NP,), jnp.float32),
    ],
)
def _sc_hist(src1d, dst1d, outdeg_out, indeg_out,
             src_v, dst_v, srow_v, drow_v, zst_v, ones_v, hs_sh, hd_sh):
    c = lax.axis_index("c")
    s = lax.axis_index("s")
    w = c * NS + s

    _fill_1d(zst_v, NSTR, 0.0)
    _fill_1d(ones_v, CH, 1.0)
    base = s * NSTR
    pltpu.sync_copy(zst_v, hs_sh.at[pl.ds(base, NSTR)])
    pltpu.sync_copy(zst_v, hd_sh.at[pl.ds(base, NSTR)])
    plsc.subcore_barrier()

    pltpu.sync_copy(src1d.at[pl.ds(w * EPW, EPW)], src_v.at[pl.ds(0, EPW)])
    pltpu.sync_copy(dst1d.at[pl.ds(w * EPW, EPW)], dst_v.at[pl.ds(0, EPW)])
    nchunks = _nchunks(w)

    @pl.when(w < TAILW)
    def _():
        pltpu.sync_copy(src1d.at[pl.ds(TAILB + w * CH, CH)],
                        src_v.at[pl.ds(EPW, CH)])
        pltpu.sync_copy(dst1d.at[pl.ds(TAILB + w * CH, CH)],
                        dst_v.at[pl.ds(EPW, CH)])

    def body(j, _):
        _copy_row(src_v, j * CH, srow_v)
        _copy_row(dst_v, j * CH, drow_v)
        pltpu.sync_copy(ones_v, hs_sh.at[srow_v], add=True)
        pltpu.sync_copy(ones_v, hd_sh.at[drow_v], add=True)
        return 0

    lax.fori_loop(0, nchunks, body, 0)
    plsc.subcore_barrier()

    pltpu.sync_copy(hs_sh.at[pl.ds(base, NSTR)],
                    outdeg_out.at[c, 0, pl.ds(base, NSTR)])
    pltpu.sync_copy(hd_sh.at[pl.ds(base, NSTR)],
                    indeg_out.at[c, 0, pl.ds(base, NSTR)])


@functools.partial(
    pl.kernel,
    out_type=jax.ShapeDtypeStruct((NC, NP, DE), jnp.float32),
    mesh=_mesh,
    scratch_types=[
        pltpu.VMEM((CH,), jnp.int32),
        pltpu.VMEM((2, CH * DE // 128, 128), jnp.float32),
        pltpu.VMEM((CH, DE), jnp.float32),
        pltpu.VMEM((NSTR, DE), jnp.float32),
        pltpu.VMEM_SHARED((NP, DE), jnp.float32),
        pltpu.SemaphoreType.DMA,
        pltpu.SemaphoreType.DMA,
        pltpu.SemaphoreType.DMA,
    ],
    compiler_params=pltpu.CompilerParams(use_tc_tiling_on_sc=False),
)
def _sc_agge(dst1d, ef128, aggh_unused, agge_out,
             drow_v, ef_v, eft_v, ze_v, agge_sh, esem0, esem1, dsem):
    c = lax.axis_index("c")
    s = lax.axis_index("s")
    w = c * NS + s
    esems = (esem0, esem1)

    def zb(i, _):
        ze_v[i, :] = jnp.zeros((DE,), jnp.float32)
        return 0

    lax.fori_loop(0, NSTR, zb, 0)
    base = s * NSTR
    pltpu.sync_copy(ze_v, agge_sh.at[pl.ds(base, NSTR)])
    plsc.subcore_barrier()

    nchunks = _nchunks(w)
    EFR = CH * DE // 128

    def issue(j, b):
        pltpu.async_copy(ef128.at[pl.ds(_ebase(w, j) // 8, EFR)],
                         ef_v.at[b], esems[b])

    def repack_ef(b):
        def rp(r, _):
            for kk in range(8):
                eft_v[8 * r + kk, :] = ef_v[b, r, pl.ds(16 * kk, 16)]
            return 0

        lax.fori_loop(0, EFR, rp, 0)

    issue(0, 0)

    def g_body(g, _):
        for b in range(2):
            j = 2 * g + b
            nb = 1 - b

            @pl.when(j < nchunks)
            def _():
                @pl.when(j + 1 < nchunks)
                def _():
                    issue(j + 1, nb)

                pltpu.async_copy(dst1d.at[pl.ds(_ebase(w, j), CH)],
                                 drow_v, dsem)
                pltpu.make_async_copy(
                    ef128.at[pl.ds(0, EFR)], ef_v.at[b], esems[b]).wait()
                repack_ef(b)
                pltpu.make_async_copy(
                    dst1d.at[pl.ds(0, CH)], drow_v, dsem).wait()
                pltpu.sync_copy(eft_v, agge_sh.at[drow_v], add=True)
        return 0

    lax.fori_loop(0, MAXCH // 2 + 1, g_body, 0)
    plsc.subcore_barrier()

    for k in range(NSTR // CH):
        pltpu.sync_copy(agge_sh.at[pl.ds(base + k * CH, CH)],
                        agge_out.at[c, pl.ds(base + k * CH, CH)])


@functools.partial(
    pl.kernel,
    out_type=jax.ShapeDtypeStruct((NC, NPA, D), jnp.float32),
    mesh=_mesh,
    scratch_types=[
        pltpu.VMEM((CH,), jnp.int32),
        pltpu.VMEM((CH,), jnp.int32),
        pltpu.VMEM((CH,), jnp.int32),
        pltpu.VMEM((CH,), jnp.int32),
        pltpu.VMEM((CH,), jnp.int32),
        pltpu.VMEM((CH,), jnp.int32),
        pltpu.VMEM((NBUF, CH, D), jnp.float32),
        pltpu.VMEM_SHARED((NPA, D), jnp.float32),
        pltpu.SemaphoreType.DMA,
        pltpu.SemaphoreType.DMA,
        pltpu.SemaphoreType.DMA,
        pltpu.SemaphoreType.DMA,
        pltpu.SemaphoreType.DMA,
        pltpu.SemaphoreType.DMA,
        pltpu.SemaphoreType.DMA,
        pltpu.SemaphoreType.DMA,
        pltpu.SemaphoreType.DMA,
    ],
)
def _sc_aggh(h_hbm, src1d, dst1d, aggh_out,
             sr0, sr1, sr2, dr0, dr1, dr2, rows_v, aggh_sh,
             is0, is1, is2, id0, id1, id2, g0, g1, g2):
    c = lax.axis_index("c")
    s = lax.axis_index("s")
    w = c * NS + s
    srs = (sr0, sr1, sr2)
    drs = (dr0, dr1, dr2)
    isems = (is0, is1, is2)
    idsems = (id0, id1, id2)
    gsems = (g0, g1, g2)

    def zb(i, _):
        for k in range(D // 16):
            rows_v[0, i, pl.ds(k * 16, 16)] = jnp.zeros((16,), jnp.float32)
        return 0

    lax.fori_loop(0, CH, zb, 0)
    base = s * NSTRA
    for k in range(NSTRA // CH):
        pltpu.sync_copy(rows_v.at[0], aggh_sh.at[pl.ds(base + k * CH, CH)])
    rem = NSTRA - (NSTRA // CH) * CH
    pltpu.sync_copy(rows_v.at[0, pl.ds(0, rem)],
                    aggh_sh.at[pl.ds(base + NSTRA - rem, rem)])
    plsc.subcore_barrier()

    nchunks = _nchunks(w)

    def issue_idx(j, b):
        eb = _ebase(w, j)
        pltpu.async_copy(src1d.at[pl.ds(eb, CH)], srs[b], isems[b])
        pltpu.async_copy(dst1d.at[pl.ds(eb, CH)], drs[b], idsems[b])

    def wait_idx(b):
        pltpu.make_async_copy(
            src1d.at[pl.ds(0, CH)], srs[b], isems[b]).wait()
        pltpu.make_async_copy(
            dst1d.at[pl.ds(0, CH)], drs[b], idsems[b]).wait()

    def issue_g(b):
        pltpu.async_copy(h_hbm.at[srs[b]], rows_v.at[b], gsems[b])

    def wait_g(b):
        pltpu.make_async_copy(
            h_hbm.at[srs[b]], rows_v.at[b], gsems[b]).wait()

    issue_idx(0, 0)
    wait_idx(0)
    issue_g(0)

    @pl.when(1 < nchunks)
    def _():
        issue_idx(1, 1)
        wait_idx(1)
        issue_g(1)

    @pl.when(2 < nchunks)
    def _():
        issue_idx(2, 2)

    def g_body(g, _):
        for b in range(NBUF):
            j = NBUF * g + b
            nxt = (b + 2) % NBUF

            @pl.when(j < nchunks)
            def _():
                @pl.when(j + 2 < nchunks)
                def _():
                    wait_idx(nxt)
                    issue_g(nxt)

                wait_g(b)
                pltpu.sync_copy(rows_v.at[b], aggh_sh.at[drs[b]], add=True)

                @pl.when(j + NBUF < nchunks)
                def _():
                    issue_idx(j + NBUF, b)
        return 0

    lax.fori_loop(0, (MAXCH + NBUF - 1) // NBUF, g_body, 0)
    plsc.subcore_barrier()

    for k in range(NSTRA // CH):
        pltpu.sync_copy(aggh_sh.at[pl.ds(base + k * CH, CH)],
                        aggh_out.at[c, pl.ds(base + k * CH, CH)])
    pltpu.sync_copy(aggh_sh.at[pl.ds(base + NSTRA - rem, rem)],
                    aggh_out.at[c, pl.ds(base + NSTRA - rem, rem)])


def _prep_body(outdeg_ref, feat_ref, h_ref):
    deg = outdeg_ref[0, 0, :N] + outdeg_ref[1, 0, :N]
    norm = lax.rsqrt(jnp.maximum(deg, 1.0))
    h_ref[...] = feat_ref[...] * norm[:, None]


def _final_body(aggh_ref, agge_ref, wh_ref, we_ref, bias_ref, indeg_ref,
                out_ref):
    aggh = aggh_ref[0, :N] + aggh_ref[1, :N]
    agge = agge_ref[0, :N] + agge_ref[1, :N]
    acc = jnp.dot(aggh, wh_ref[...], preferred_element_type=jnp.float32)
    acc = acc + jnp.dot(agge, we_ref[...], preferred_element_type=jnp.float32)
    indeg = indeg_ref[0, 0, :N] + indeg_ref[1, 0, :N]
    norm = lax.rsqrt(jnp.maximum(indeg, 1.0))
    out_ref[...] = acc * norm[:, None] + bias_ref[...]


@jax.jit
def kernel(feat, edge_index, edge_feat, weight, bias):
    src1d, dst1d = _sc_split(edge_index)
    ef128 = edge_feat.reshape(E * DE // 128, 128)
    outdeg_p, indeg_p = _sc_hist(src1d, dst1d)

    h = pl.pallas_call(
        _prep_body,
        out_shape=jax.ShapeDtypeStruct((N, D), jnp.float32),
    )(outdeg_p, feat)

    aggh_p = _sc_aggh(h, src1d, dst1d)
    agge_p = _sc_agge(dst1d, ef128, aggh_p)

    out = pl.pallas_call(
        _final_body,
        out_shape=jax.ShapeDtypeStruct((N, DO), jnp.float32),
    )(aggh_p, agge_p, weight[:D], weight[D:], bias.reshape(1, DO), indeg_p)
    return out
